# baseline (speedup 1.0000x reference)
import functools

import jax
import jax.numpy as jnp
from jax import lax
from jax.experimental import pallas as pl
from jax.experimental.pallas import tpu as pltpu
from jax.experimental.pallas import tpu_sc as plsc


@functools.cache
def _build(batch: int, hist: int, vocab: int, d: int):
  info = plsc.get_sparse_core_info()
  nc, ns = info.num_cores, info.num_subcores
  nw = nc * ns
  per_w = batch // nw
  grp = 4
  nbuf = 2 * grp
  steps = per_w // nbuf
  assert batch == nw * per_w and per_w % nbuf == 0

  mesh = plsc.VectorSubcoreMesh(core_axis_name="c", subcore_axis_name="s")

  def body(idx_hbm, table_hbm, out_hbm, idx_v, bufs, gsems, wsems):
    wid = lax.axis_index("s") * nc + lax.axis_index("c")
    obase = wid * per_w

    pltpu.sync_copy(idx_hbm.at[pl.ds(obase, per_w)], idx_v)

    def gwait(b):
      pltpu.make_async_copy(table_hbm.at[idx_v.at[0]], bufs[b], gsems[b]).wait()

    def wwait(b):
      pltpu.make_async_copy(bufs[b], out_hbm.at[obase], wsems[b]).wait()

    for b in range(grp):
      pltpu.async_copy(table_hbm.at[idx_v.at[b]], bufs[b], gsems[b])

    def step(i, carry):
      j0 = nbuf * i
      for h in range(2):
        jh = j0 + grp * h
        for k in range(grp):
          b = grp * h + k
          gwait(b)
          pltpu.async_copy(bufs[b], out_hbm.at[obase + jh + k], wsems[b])

        def issue_next(jh=jh, h=h, guard_w=True):
          for k in range(grp):
            b = grp * (1 - h) + k

            if guard_w:
              wwait(b)

            pltpu.async_copy(
                table_hbm.at[idx_v.at[jh + grp + k]], bufs[b], gsems[b])

        if h == 0:
          @pl.when(i > 0)
          def _():
            issue_next(guard_w=True)

          @pl.when(i == 0)
          def _():
            issue_next(guard_w=False)
        else:
          @pl.when(i < steps - 1)
          def _():
            issue_next(guard_w=True)

      return carry

    lax.fori_loop(0, steps, step, 0)

    for b in range(nbuf):
      wwait(b)

  return pl.kernel(
      body,
      out_type=jax.ShapeDtypeStruct((batch, hist, d), jnp.float32),
      mesh=mesh,
      scratch_types=[
          pltpu.VMEM((per_w, hist), jnp.int32),
          [pltpu.VMEM((hist, d), jnp.float32) for _ in range(nbuf)],
          [pltpu.SemaphoreType.DMA for _ in range(nbuf)],
          [pltpu.SemaphoreType.DMA for _ in range(nbuf)],
      ],
  )


@jax.jit
def kernel(x, table):
  b, h = x.shape
  vocab, d = table.shape
  return _build(b, h, vocab, d)(x.astype(jnp.int32), table)
